# baseline (speedup 1.0000x reference)
import functools

import jax
import jax.numpy as jnp
from jax import lax
from jax.experimental import pallas as pl
from jax.experimental.pallas import tpu as pltpu
from jax.experimental.pallas import tpu_sc as plsc

VOCAB = 8192
DIM = 32
COMMIT = 0.25
N_POINTS = 8192


_BN = 256


def _argmin_body(x_ref, w_ref, x2_ref, w2_ref, idx_ref):
    x = x_ref[...]
    w = w_ref[...]
    x2 = x2_ref[...]
    w2 = w2_ref[...]
    xw = lax.dot_general(
        x, w, (((1,), (1,)), ((), ())),
        preferred_element_type=jnp.float32,
    )
    d2 = x2 + w2[None, :] - 2.0 * xw
    d = jnp.sqrt(jnp.maximum(d2, 0.0))
    m = jnp.min(d, axis=1, keepdims=True)
    ks = lax.broadcasted_iota(jnp.int32, d.shape, 1)
    idx = jnp.min(jnp.where(d == m, ks, jnp.int32(VOCAB)), axis=1)
    idx_ref[...] = idx


def _tc_argmin(xf, W, x2, w2):
    return pl.pallas_call(
        _argmin_body,
        grid=(N_POINTS // _BN,),
        in_specs=[
            pl.BlockSpec((_BN, DIM), lambda i: (i, 0)),
            pl.BlockSpec((VOCAB, DIM), lambda i: (0, 0)),
            pl.BlockSpec((_BN, 1), lambda i: (i, 0)),
            pl.BlockSpec((VOCAB,), lambda i: (0,)),
        ],
        out_specs=pl.BlockSpec((_BN,), lambda i: (i,)),
        out_shape=jax.ShapeDtypeStruct((N_POINTS,), jnp.int32),
    )(xf, W, x2, w2)


_NC, _NS, _L = 2, 16, 16
_NW = _NC * _NS
_BPW = N_POINTS // _NW
_CHUNK = 128


def _sc_body(w_hbm, idx_hbm, x_hbm, qst_hbm, part_hbm,
             idx_v, rows_v, x_v, acc_v, sem):
    wid = lax.axis_index("s") * _NC + lax.axis_index("c")
    base = wid * _BPW
    for j in range(_BPW // _CHUNK):
        pltpu.sync_copy(idx_hbm.at[pl.ds(base + j * _CHUNK, _CHUNK)],
                        idx_v.at[j])
    for j in range(_BPW // _CHUNK):
        pltpu.async_copy(
            w_hbm.at[idx_v.at[j]],
            rows_v.at[pl.ds(j * _CHUNK, _CHUNK)],
            sem,
        ).wait()
    pltpu.sync_copy(x_hbm.at[pl.ds(base, _BPW)], x_v)

    def body(i, acc):
        a = acc
        for h in range(0, DIM, _L):
            q = rows_v[i, pl.ds(h, _L)]
            xx = x_v[i, pl.ds(h, _L)]
            t = q - xx
            rows_v[i, pl.ds(h, _L)] = xx + t
            a = a + t * t
        return a

    acc = lax.fori_loop(0, _BPW, body, jnp.zeros((_L,), jnp.float32))
    acc_v[...] = acc
    pltpu.sync_copy(acc_v, part_hbm.at[wid])
    pltpu.sync_copy(rows_v, qst_hbm.at[pl.ds(base, _BPW)])


@functools.cache
def _sc_gather():
    return pl.kernel(
        _sc_body,
        out_type=[
            jax.ShapeDtypeStruct((N_POINTS, DIM), jnp.float32),
            jax.ShapeDtypeStruct((_NW, _L), jnp.float32),
        ],
        mesh=plsc.VectorSubcoreMesh(core_axis_name="c", subcore_axis_name="s",
                                    num_cores=_NC, num_subcores=_NS),
        scratch_types=[
            pltpu.VMEM((_BPW // _CHUNK, _CHUNK), jnp.int32),
            pltpu.VMEM((_BPW, DIM), jnp.float32),
            pltpu.VMEM((_BPW, DIM), jnp.float32),
            pltpu.VMEM((_L,), jnp.float32),
            pltpu.SemaphoreType.DMA,
        ],
        compiler_params=pltpu.CompilerParams(use_tc_tiling_on_sc=False),
    )


def kernel(x, W):
    xf = x.reshape(N_POINTS, DIM)
    x2 = jnp.sum(x * x, axis=-1, keepdims=True).reshape(N_POINTS, 1)
    w2 = jnp.sum(W * W, axis=-1)
    idx = _tc_argmin(xf, W, x2, w2)
    qst, parts = _sc_gather()(W, idx, xf)
    c = jnp.sum(parts) * jnp.float32(1.0 / (N_POINTS * DIM))
    codebook_loss = c
    loss = codebook_loss + jnp.float32(COMMIT) * c
    return (
        qst.reshape(x.shape),
        loss,
        codebook_loss,
        idx.reshape(x.shape[:2]),
    )


# speedup vs baseline: 1.3232x; 1.3232x over previous
import functools

import jax
import jax.numpy as jnp
from jax import lax
from jax.experimental import pallas as pl
from jax.experimental.pallas import tpu as pltpu
from jax.experimental.pallas import tpu_sc as plsc

VOCAB = 8192
DIM = 32
COMMIT = 0.25
N_POINTS = 8192


_BN = 256


def _argmin_body(x_ref, w_ref, x2_ref, w2_ref, idx_ref):
    x2d = x_ref[...]
    w = w_ref[...]
    x2 = x2_ref[...]
    w2 = w2_ref[...]
    xw2 = lax.dot_general(
        x2d, w, (((1,), (1,)), ((), ())),
        preferred_element_type=jnp.float32,
    )
    d2 = x2 + w2[None, :] - xw2
    d = jnp.sqrt(jnp.maximum(d2, 0.0))
    m = jnp.min(d, axis=1, keepdims=True)
    ks = lax.broadcasted_iota(jnp.int32, d.shape, 1)
    idx = jnp.min(jnp.where(d == m, ks, jnp.int32(VOCAB)), axis=1)
    idx_ref[...] = idx


def _tc_argmin(xf, W, x2, w2):
    return pl.pallas_call(
        _argmin_body,
        grid=(N_POINTS // _BN,),
        in_specs=[
            pl.BlockSpec((_BN, DIM), lambda i: (i, 0)),
            pl.BlockSpec((VOCAB, DIM), lambda i: (0, 0)),
            pl.BlockSpec((_BN, 1), lambda i: (i, 0)),
            pl.BlockSpec((VOCAB,), lambda i: (0,)),
        ],
        out_specs=pl.BlockSpec((_BN,), lambda i: (i,)),
        out_shape=jax.ShapeDtypeStruct((N_POINTS,), jnp.int32),
    )(xf, W, x2, w2)


_NC, _NS, _L = 2, 16, 16
_NW = _NC * _NS
_BPW = N_POINTS // _NW
_CHUNK = 128


def _sc_body(w_hbm, idx_hbm, x_hbm, qst_hbm, part_hbm,
             idx_v, rows_v, x_v, acc_v, sem):
    wid = lax.axis_index("s") * _NC + lax.axis_index("c")
    base = wid * _BPW
    for j in range(_BPW // _CHUNK):
        pltpu.sync_copy(idx_hbm.at[pl.ds(base + j * _CHUNK, _CHUNK)],
                        idx_v.at[j])
    for j in range(_BPW // _CHUNK):
        pltpu.async_copy(
            w_hbm.at[idx_v.at[j]],
            rows_v.at[pl.ds(j * _CHUNK, _CHUNK)],
            sem,
        ).wait()
    pltpu.sync_copy(x_hbm.at[pl.ds(base, _BPW)], x_v)

    def body(i, acc):
        a = acc
        for h in range(0, DIM, _L):
            q = rows_v[i, pl.ds(h, _L)]
            xx = x_v[i, pl.ds(h, _L)]
            t = q - xx
            rows_v[i, pl.ds(h, _L)] = xx + t
            a = a + t * t
        return a

    acc = lax.fori_loop(0, _BPW, body, jnp.zeros((_L,), jnp.float32))
    acc_v[...] = acc
    pltpu.sync_copy(acc_v, part_hbm.at[wid])
    pltpu.sync_copy(rows_v, qst_hbm.at[pl.ds(base, _BPW)])


@functools.cache
def _sc_gather():
    return pl.kernel(
        _sc_body,
        out_type=[
            jax.ShapeDtypeStruct((N_POINTS, DIM), jnp.float32),
            jax.ShapeDtypeStruct((_NW, _L), jnp.float32),
        ],
        mesh=plsc.VectorSubcoreMesh(core_axis_name="c", subcore_axis_name="s",
                                    num_cores=_NC, num_subcores=_NS),
        scratch_types=[
            pltpu.VMEM((_BPW // _CHUNK, _CHUNK), jnp.int32),
            pltpu.VMEM((_BPW, DIM), jnp.float32),
            pltpu.VMEM((_BPW, DIM), jnp.float32),
            pltpu.VMEM((_L,), jnp.float32),
            pltpu.SemaphoreType.DMA,
        ],
        compiler_params=pltpu.CompilerParams(use_tc_tiling_on_sc=False),
    )


def kernel(x, W):
    xf = x.reshape(N_POINTS, DIM)
    x2 = jnp.sum(x * x, axis=-1, keepdims=True).reshape(N_POINTS, 1)
    w2 = jnp.sum(W * W, axis=-1)
    idx = _tc_argmin(xf + xf, W, x2, w2)
    qst, parts = _sc_gather()(W, idx, xf)
    c = jnp.sum(parts) * jnp.float32(1.0 / (N_POINTS * DIM))
    codebook_loss = c
    loss = codebook_loss + jnp.float32(COMMIT) * c
    return (
        qst.reshape(x.shape),
        loss,
        codebook_loss,
        idx.reshape(x.shape[:2]),
    )
